# baseline (speedup 1.0000x reference)
import functools

import jax
import jax.numpy as jnp
from jax import lax

jax.config.update('jax_default_matmul_precision', 'float32')
from jax.experimental import pallas as pl
from jax.experimental.pallas import tpu as pltpu
from jax.experimental.pallas import tpu_sc as plsc

NC = 2
NS = 16
L = 16
NW = NC * NS
B = 128

F32 = jnp.float32
I32 = jnp.int32


def _pad_to(x, m):
    return ((x + m - 1) // m) * m


def _zero_chunk(rpt):
    for c in (128, 64, 40, 32, 16, 8):
        if rpt % c == 0:
            return c
    return 1


@functools.lru_cache(maxsize=None)
def _make_gather_rot(n_nodes, e_pad, cin):
    nb = e_pad // (NW * B)
    w = 5 * cin
    d_list = (0, 8) if cin == 24 else (0, 16, 32)
    mesh = plsc.VectorSubcoreMesh(core_axis_name="c", subcore_axis_name="s")

    @functools.partial(
        pl.kernel,
        out_type=jax.ShapeDtypeStruct((e_pad, w), F32),
        mesh=mesh,
        compiler_params=pltpu.CompilerParams(use_tc_tiling_on_sc=False,
                                             needs_layout_passes=False),
        scratch_types=[
            pltpu.VMEM((B,), I32),
            pltpu.VMEM((2 * B,), F32),
            pltpu.VMEM((B, w), F32),
            pltpu.SemaphoreType.DMA,
        ],
    )
    def gather_rot(x_hbm, src_hbm, cs_hbm, out_hbm, srcv, coefv, gv, sem):
        cid = lax.axis_index("c")
        sid = lax.axis_index("s")
        wid = cid * NS + sid
        ebase = wid * (nb * B)

        def batch(b, _):
            off = ebase + b * B
            pltpu.sync_copy(src_hbm.at[pl.ds(off, B)], srcv)
            for k in range(2):
                pltpu.sync_copy(cs_hbm.at[pl.ds(k * e_pad + off, B)],
                                coefv.at[pl.ds(k * B, B)])
            pltpu.async_copy(x_hbm.at[srcv], gv, sem).wait()

            def ebody(e, _):
                def bl(k):
                    return plsc.load_gather(
                        coefv, [jnp.full((L,), k * B + e, I32)])
                c = bl(0)
                s = bl(1)
                c2 = c * c - s * s
                s2 = 2.0 * c * s
                zs = {}
                for d in d_list:
                    for k in (1, 2, 3, 4):
                        zs[(d, k)] = gv[e, pl.ds(k * cin + d, L)]
                for d in d_list:
                    def st(pos, v):
                        gv[e, pl.ds(pos + d, L)] = v
                    st(cin, c * zs[(d, 1)] - s * zs[(d, 2)])
                    st(2 * cin, s * zs[(d, 1)] + c * zs[(d, 2)])
                    st(3 * cin, c2 * zs[(d, 3)] - s2 * zs[(d, 4)])
                    st(4 * cin, s2 * zs[(d, 3)] + c2 * zs[(d, 4)])
                return 0
            lax.fori_loop(0, B, ebody, 0)
            pltpu.sync_copy(gv, out_hbm.at[pl.ds(off, B)])
            return 0
        lax.fori_loop(0, nb, batch, 0)

    return gather_rot


@functools.lru_cache(maxsize=None)
def _make_segsum(n_pad, e_pad):
    nb = e_pad // (NW * B)
    rpt = n_pad // NS
    zc = _zero_chunk(rpt)
    mesh = plsc.VectorSubcoreMesh(core_axis_name="c", subcore_axis_name="s")

    @functools.partial(
        pl.kernel,
        out_type=jax.ShapeDtypeStruct((2 * n_pad, 120), F32),
        mesh=mesh,
        compiler_params=pltpu.CompilerParams(use_tc_tiling_on_sc=False,
                                             needs_layout_passes=False),
        scratch_types=[
            pltpu.VMEM((B,), I32),
            pltpu.VMEM((B, 120), F32),
            pltpu.VMEM_SHARED((n_pad, 120), F32),
            pltpu.SemaphoreType.DMA,
        ],
    )
    def segsum(m_hbm, dst_hbm, zer_hbm, out_hbm, dstv, mv, accum, sem):
        cid = lax.axis_index("c")
        sid = lax.axis_index("s")
        wid = cid * NS + sid

        pltpu.sync_copy(zer_hbm.at[pl.ds(0, zc)], mv.at[pl.ds(0, zc)])
        row0 = sid * rpt

        def zbody(j, _):
            pltpu.sync_copy(mv.at[pl.ds(0, zc)],
                            accum.at[pl.ds(row0 + j * zc, zc)])
            return 0
        lax.fori_loop(0, rpt // zc, zbody, 0)
        plsc.subcore_barrier()

        ebase = wid * (nb * B)

        def batch(b, _):
            off = ebase + b * B
            pltpu.sync_copy(dst_hbm.at[pl.ds(off, B)], dstv)
            pltpu.sync_copy(m_hbm.at[pl.ds(off, B)], mv)
            pltpu.sync_copy(mv, accum.at[dstv], add=True)
            return 0
        lax.fori_loop(0, nb, batch, 0)
        plsc.subcore_barrier()

        def fbody(j, _):
            r = row0 + j * zc
            pltpu.sync_copy(accum.at[pl.ds(r, zc)], mv.at[pl.ds(0, zc)])
            pltpu.sync_copy(mv.at[pl.ds(0, zc)],
                            out_hbm.at[pl.ds(cid * n_pad + r, zc)])
            return 0
        lax.fori_loop(0, rpt // zc, fbody, 0)

    return segsum


@functools.lru_cache(maxsize=None)
def _make_pool_pass(nf_pad, nc_pad):
    nb = nf_pad // (NW * B)
    rpt = nc_pad // NS
    zc = _zero_chunk(rpt)
    mesh = plsc.VectorSubcoreMesh(core_axis_name="c", subcore_axis_name="s")

    @functools.partial(
        pl.kernel,
        out_type=jax.ShapeDtypeStruct((2 * nc_pad, 128), F32),
        mesh=mesh,
        compiler_params=pltpu.CompilerParams(use_tc_tiling_on_sc=False, needs_layout_passes=False),
        scratch_types=[
            pltpu.VMEM((B,), I32),
            pltpu.VMEM((B, 128), F32),
            pltpu.VMEM_SHARED((nc_pad, 128), F32),
            pltpu.SemaphoreType.DMA,
        ],
    )
    def pool_pass(xr_hbm, cl_hbm, zer_hbm, out_hbm, clv, xv, accum, sem):
        cid = lax.axis_index("c")
        sid = lax.axis_index("s")
        wid = cid * NS + sid

        pltpu.sync_copy(zer_hbm.at[pl.ds(0, zc)], xv.at[pl.ds(0, zc)])
        row0 = sid * rpt

        def zbody(j, _):
            pltpu.sync_copy(xv.at[pl.ds(0, zc)],
                            accum.at[pl.ds(row0 + j * zc, zc)])
            return 0
        lax.fori_loop(0, rpt // zc, zbody, 0)
        plsc.subcore_barrier()

        base = wid * (nb * B)

        def batch(b, _):
            off = base + b * B
            pltpu.sync_copy(cl_hbm.at[pl.ds(off, B)], clv)
            pltpu.sync_copy(xr_hbm.at[pl.ds(off, B)], xv)
            pltpu.sync_copy(xv, accum.at[clv], add=True)
            return 0
        lax.fori_loop(0, nb, batch, 0)
        plsc.subcore_barrier()

        def fbody(j, _):
            r = row0 + j * zc
            pltpu.sync_copy(accum.at[pl.ds(r, zc)], xv.at[pl.ds(0, zc)])
            pltpu.sync_copy(xv.at[pl.ds(0, zc)],
                            out_hbm.at[pl.ds(cid * nc_pad + r, zc)])
            return 0
        lax.fori_loop(0, rpt // zc, fbody, 0)

    return pool_pass


@functools.lru_cache(maxsize=None)
def _make_unpool_pass(nc, nf_pad):
    nb = nf_pad // (NW * B)
    mesh = plsc.VectorSubcoreMesh(core_axis_name="c", subcore_axis_name="s")

    @functools.partial(
        pl.kernel,
        out_type=jax.ShapeDtypeStruct((nf_pad, 120), F32),
        mesh=mesh,
        compiler_params=pltpu.CompilerParams(use_tc_tiling_on_sc=False, needs_layout_passes=False),
        scratch_types=[
            pltpu.VMEM((B,), I32),
            pltpu.VMEM((B, 120), F32),
            pltpu.SemaphoreType.DMA,
        ],
    )
    def unpool_pass(xc_hbm, cl_hbm, out_hbm, clv, xv, sem):
        cid = lax.axis_index("c")
        sid = lax.axis_index("s")
        wid = cid * NS + sid
        base = wid * (nb * B)

        def batch(b, _):
            off = base + b * B
            pltpu.sync_copy(cl_hbm.at[pl.ds(off, B)], clv)
            pltpu.async_copy(xc_hbm.at[clv], xv, sem).wait()
            pltpu.sync_copy(xv, out_hbm.at[pl.ds(off, B)])
            return 0
        lax.fori_loop(0, nb, batch, 0)

    return unpool_pass


def _rotate_im(z, c, s):
    c = c[:, None]
    s = s[:, None]
    c2 = c * c - s * s
    s2 = 2.0 * c * s
    return jnp.stack([
        z[:, 0],
        c * z[:, 1] - s * z[:, 2],
        s * z[:, 1] + c * z[:, 2],
        c2 * z[:, 3] - s2 * z[:, 4],
        s2 * z[:, 3] + c2 * z[:, 4],
    ], axis=1)


def _nonlin_im(x, b):
    s0 = jax.nn.relu(x[:, 0] + b[None, :])
    m1 = jnp.sqrt(x[:, 1] ** 2 + x[:, 2] ** 2 + 1e-12)
    g1 = jax.nn.relu(m1 + b[None, :]) / (m1 + 1e-6)
    m2 = jnp.sqrt(x[:, 3] ** 2 + x[:, 4] ** 2 + 1e-12)
    g2 = jax.nn.relu(m2 + b[None, :]) / (m2 + 1e-6)
    return jnp.stack([s0, g1 * x[:, 1], g1 * x[:, 2], g2 * x[:, 3], g2 * x[:, 4]], axis=1)


def _gem_conv_sc(x, lv, W, pc):
    n, n_pad, e, e_pad = lv['n'], lv['n_pad'], lv['e'], lv['e_pad']
    cin = x.shape[2]
    gcin = 24 if cin == 8 else cin
    if cin == 8:
        x = jnp.zeros((n, 5, 24), F32).at[:, :, :8].set(x)
    gather = _make_gather_rot(n, e_pad, gcin)
    msg = gather(x.reshape(n, 5 * gcin), lv['src'], lv['cs'])
    msg = msg[:e].reshape(e, 5, gcin)[:, :, :cin]
    m = jnp.einsum('eci,er,rco->eoi', msg.transpose(0, 2, 1), pc, W)
    m = m.transpose(0, 2, 1)
    mp = jnp.zeros((e_pad, 120), F32).at[:e].set(m.reshape(e, 120))
    segsum = _make_segsum(n_pad, e_pad)
    out = segsum(mp, lv['dst'], lv['zer120'])
    return (out[:n_pad] + out[n_pad:])[:n].reshape(n, 5, 24)


def _res_block_sc(x, lv, p):
    h = _nonlin_im(_gem_conv_sc(x, lv, p['W1'], lv['pc']), p['b1'])
    h = _gem_conv_sc(h, lv, p['W2'], lv['pc'])
    sc = jnp.einsum('nci,co->noi', x.transpose(0, 2, 1), p['Wres']).transpose(0, 2, 1)
    return _nonlin_im(h + sc, p['b2'])


def _prep_level(ei, pc, cn, n):
    e = ei.shape[1]
    e_pad = _pad_to(e, NW * B)
    n_pad = _pad_to(n, NS * 128)
    src = jnp.pad(ei[0], (0, e_pad - e))
    dst = jnp.pad(ei[1], (0, e_pad - e))
    cs = jnp.concatenate([
        jnp.pad(cn[:, 0], (0, e_pad - e)),
        jnp.pad(cn[:, 1], (0, e_pad - e)),
    ])
    return {
        'n': n, 'n_pad': n_pad, 'e': e, 'e_pad': e_pad,
        'src': src, 'dst': dst, 'cs': cs, 'pc': pc,
        'zer120': jnp.zeros((B, 120), F32),
    }


def _pool_sc(x, cluster, tr, n_coarse):
    n = x.shape[0]
    nf_pad = _pad_to(n, NW * B)
    nc_pad = _pad_to(n_coarse, NS * 128)
    xr = _rotate_im(x, tr[:, 0], tr[:, 1]).reshape(n, 120)
    xrp = jnp.zeros((nf_pad, 128), F32)
    xrp = xrp.at[:n, :120].set(xr).at[:n, 120].set(1.0)
    clp = jnp.pad(cluster, (0, nf_pad - n))
    pool = _make_pool_pass(nf_pad, nc_pad)
    out = pool(xrp, clp, jnp.zeros((B, 128), F32))
    s = (out[:nc_pad] + out[nc_pad:])[:n_coarse]
    cnt = jnp.maximum(s[:, 120:121], 1.0)
    return (s[:, :120] / cnt).reshape(n_coarse, 5, 24)


def _unpool_sc(x, cluster, tr):
    n_c = x.shape[0]
    n_f = cluster.shape[0]
    nf_pad = _pad_to(n_f, NW * B)
    clp = jnp.pad(cluster, (0, nf_pad - n_f))
    unpool = _make_unpool_pass(n_c, nf_pad)
    g = unpool(x.reshape(n_c, 120), clp)[:n_f].reshape(n_f, 5, 24)
    return _rotate_im(g, tr[:, 0], -tr[:, 1])


def _head_kernel(fv_ref, o_ref):
    fv = fv_ref[...]
    v0 = fv[:, 8:9]
    v1 = fv[:, 9:10]
    o = fv[:, 0:3] * v0 + fv[:, 3:6] * v1
    o_ref[...] = jnp.pad(o, ((0, 0), (0, 125)))


def kernel(features, geo, edge_index0, edge_index1, edge_index2, precomp0, precomp1, precomp2, connection0, connection1, connection2, cluster1, cluster2, transport1, transport2, frame, params):
    N0, N1, N2 = 10000, 2500, 625
    lv0 = _prep_level(edge_index0, precomp0, connection0, N0)
    lv1 = _prep_level(edge_index1, precomp1, connection1, N1)
    lv2 = _prep_level(edge_index2, precomp2, connection2, N2)

    x = jnp.concatenate(
        [jnp.transpose(features, (0, 2, 1)),
         jnp.zeros((N0, 5, 1), F32).at[:, 0, 0].set(geo)], axis=2)
    x = _res_block_sc(x, lv0, params['c01'])
    x = _res_block_sc(x, lv0, params['c02'])
    copy0 = x
    x = _pool_sc(x, cluster1, transport1, N1)
    x = _res_block_sc(x, lv1, params['c11'])
    x = _res_block_sc(x, lv1, params['c12'])
    copy1 = x
    x = _pool_sc(x, cluster2, transport2, N2)
    x = _res_block_sc(x, lv2, params['c21'])
    x = _res_block_sc(x, lv2, params['c22'])
    x = _unpool_sc(x, cluster2, transport2)
    x = jnp.concatenate([x, copy1], axis=2)
    x = _res_block_sc(x, lv1, params['c13'])
    x = _res_block_sc(x, lv1, params['c14'])
    x = _res_block_sc(x, lv1, params['c15'])
    x = _res_block_sc(x, lv1, params['c16'])
    x = _unpool_sc(x, cluster1, transport1)
    x = jnp.concatenate([x, copy0], axis=2)
    x = _res_block_sc(x, lv0, params['c03'])
    x = _res_block_sc(x, lv0, params['c04'])
    x = _res_block_sc(x, lv0, params['c05'])
    p06 = params['c06']
    p06p = {
        'W1': jnp.pad(p06['W1'], ((0, 0), (0, 0), (0, 23))),
        'W2': jnp.pad(p06['W2'], ((0, 0), (0, 23), (0, 23))),
        'Wres': jnp.pad(p06['Wres'], ((0, 0), (0, 23))),
        'b1': jnp.pad(p06['b1'], (0, 23)),
        'b2': jnp.pad(p06['b2'], (0, 23)),
    }
    x = _res_block_sc(x, lv0, p06p)

    v = x[:, 1:3, 0]
    return jnp.einsum('nij,nj->ni', frame, v)
